# baseline (speedup 1.0000x reference)
import jax
import jax.numpy as jnp
from jax import lax
from jax.experimental import pallas as pl
from jax.experimental.pallas import tpu as pltpu
from jax.experimental.pallas import tpu_sc as plsc

NUM_NODES = 1000000
E = 128
B = 16384
S = 6
L = 16
NC = 2
NS = 16
NW = NC * NS
B_PER_W = B // NW
CH = 64
NCHUNK = B_PER_W // CH
SIDX_ROWS = CH * S // E


def _logits_body(src_hbm, smp_hbm, node_hbm, ctx_hbm, out_hbm,
                 src_idx, smp_idx, src_rows, smp_rows, out_v, stage, sems):
    wid = lax.axis_index("s") * NC + lax.axis_index("c")
    base = wid * B_PER_W
    lane = lax.iota(jnp.int32, L)

    def mk_copies(c, p):
        cps = [pltpu.make_async_copy(
            node_hbm.at[src_idx.at[p]], src_rows.at[p], sems.at[p])]
        for j in range(SIDX_ROWS):
            cps.append(pltpu.make_async_copy(
                ctx_hbm.at[smp_idx.at[p, pl.ds(j * E, E)]],
                smp_rows.at[p, pl.ds(j * E, E)], sems.at[p]))
        return cps

    def stage_and_gather(c, p):
        b0 = base + c * CH
        pltpu.sync_copy(src_hbm.at[pl.ds(b0, CH)], src_idx.at[p])
        pltpu.sync_copy(smp_hbm.at[pl.ds(b0 * S, CH * S)], smp_idx.at[p])
        cps = mk_copies(c, p)
        for cp in cps:
            cp.start()
        return cps

    GB = 8
    GR = GB * S

    def tree_sum(vs):
        while len(vs) > 1:
            vs = [a + b for a, b in zip(vs[::2], vs[1::2])]
        return vs[0]

    def compute_chunk(c, p):
        def g_body(g, _):
            b0 = g * GB
            for k in range(GB):
                bb = b0 + k
                sv = [src_rows[p, bb, pl.ds(j * L, L)]
                      for j in range(E // L)]
                for s in range(S):
                    row = bb * S + s
                    prods = [sv[j] * smp_rows[p, row, pl.ds(j * L, L)]
                             for j in range(E // L)]
                    stage[k * S + s, :] = tree_sum(prods)
            for t in range(GR // L):
                ridx = t * L + lane
                gs = [plsc.load_gather(stage, [ridx, lane * 0 + j])
                      for j in range(L)]
                out_v[pl.ds(g * GR + t * L, L)] = tree_sum(gs)
            return 0

        lax.fori_loop(0, CH // GB, g_body, 0)
        pltpu.sync_copy(out_v, out_hbm.at[pl.ds((base + c * CH) * S, CH * S)])

    cps = [stage_and_gather(0, 0), None]

    def pair_body(h, _):
        c0 = 2 * h
        cps[1] = stage_and_gather(c0 + 1, 1)
        for cp in cps[0]:
            cp.wait()
        compute_chunk(c0, 0)

        @pl.when(h < NCHUNK // 2 - 1)
        def _():
            stage_and_gather(c0 + 2, 0)
        cps[0] = mk_copies(c0 + 2, 0)
        for cp in cps[1]:
            cp.wait()
        compute_chunk(c0 + 1, 1)
        return 0

    lax.fori_loop(0, NCHUNK // 2, pair_body, 0)


def _sc_logits(source_nodes, sample_rows_idx, node_embedding,
               context_embedding):
    mesh = plsc.VectorSubcoreMesh(
        core_axis_name="c", subcore_axis_name="s",
        num_cores=NC, num_subcores=NS)
    return pl.kernel(
        _logits_body,
        out_type=jax.ShapeDtypeStruct((B * S,), jnp.float32),
        mesh=mesh,
        scratch_types=[
            pltpu.VMEM((2, CH), jnp.int32),
            pltpu.VMEM((2, CH * S), jnp.int32),
            pltpu.VMEM((2, CH, E), jnp.float32),
            pltpu.VMEM((2, CH * S, E), jnp.float32),
            pltpu.VMEM((CH * S,), jnp.float32),
            pltpu.VMEM((8 * S, L), jnp.float32),
            pltpu.SemaphoreType.DMA((2,)),
        ],
        compiler_params=pltpu.CompilerParams(needs_layout_passes=False),
    )(source_nodes, sample_rows_idx, node_embedding, context_embedding)


def _loss_body(lg_ref, out_ref):
    x = lg_ref[...]
    r = lax.broadcasted_iota(jnp.int32, x.shape, 0)
    c = lax.broadcasted_iota(jnp.int32, x.shape, 1)
    is_pos = ((r * E + c) % S) == 0
    t = jnp.where(is_pos, x, -x)
    sp = jnp.maximum(-t, 0.0) + jnp.log1p(jnp.exp(-jnp.abs(t)))
    w = jnp.where(is_pos, 1.0 / B, 1.0 / (B * (S - 1)))
    out_ref[0, 0] = jnp.sum(sp * w)


def _tc_loss(logits2d):
    return pl.pallas_call(
        _loss_body,
        out_shape=jax.ShapeDtypeStruct((1, 1), jnp.float32),
        out_specs=pl.BlockSpec(memory_space=pltpu.SMEM),
    )(logits2d)


def kernel(source_nodes, sample_nodes, node_embedding, context_embedding):
    src = jnp.asarray(source_nodes, jnp.int32)
    smp = jnp.asarray(sample_nodes, jnp.int32).reshape(B * S)
    logits = _sc_logits(src, smp, node_embedding, context_embedding)
    loss = _tc_loss(logits.reshape(B * S // E, E))
    return loss[0, 0]
